# baseline (speedup 1.0000x reference)
import functools

import jax
import jax.numpy as jnp
from jax import lax
from jax.experimental import pallas as pl
from jax.experimental.pallas import tpu as pltpu
from jax.experimental.pallas import tpu_sc as plsc

B = 16384
V = 1000000
D = 64

NC, NS = 2, 16
NW = NC * NS
BPW = B // NW
ICH = 128
NJ = BPW // ICH


def _sc_gather_body(idxa1_hbm, idxb1_hbm, idxa2_hbm, idxb2_hbm,
                    ua_hbm, ub_hbm, ia_hbm, ib_hbm,
                    ga1_hbm, gb1_hbm, ga2_hbm, gb2_hbm,
                    idxv, rows, sem):
    wid = lax.axis_index("s") * NC + lax.axis_index("c")
    base = wid * BPW
    for src, tbl, dst in ((idxa1_hbm, ua_hbm, ga1_hbm),
                          (idxb1_hbm, ub_hbm, gb1_hbm),
                          (idxa2_hbm, ia_hbm, ga2_hbm),
                          (idxb2_hbm, ib_hbm, gb2_hbm)):
        pltpu.sync_copy(src.at[wid], idxv)
        copies = [
            pltpu.async_copy(tbl.at[idxv.at[j]],
                             rows.at[pl.ds(j * ICH, ICH)], sem)
            for j in range(NJ)
        ]
        for c in copies:
            c.wait()
        pltpu.sync_copy(rows, dst.at[pl.ds(base, BPW)])


@functools.lru_cache(maxsize=None)
def _sc_gather():
    return pl.kernel(
        _sc_gather_body,
        out_type=tuple(
            jax.ShapeDtypeStruct((B, 2 * D), jnp.float32) for _ in range(4)),
        mesh=plsc.VectorSubcoreMesh(core_axis_name="c", subcore_axis_name="s"),
        scratch_types=[
            pltpu.VMEM((NJ, ICH), jnp.int32),
            pltpu.VMEM((BPW, 2 * D), jnp.float32),
            pltpu.SemaphoreType.DMA,
        ],
    )


BLK = 2048


def _sel(ga_ref, gb_ref, hb, p):
    lo = ga_ref[:, :D] * (1.0 - hb) + gb_ref[:, :D] * hb
    hi = ga_ref[:, D:] * (1.0 - hb) + gb_ref[:, D:] * hb
    return lo * (1.0 - p) + hi * p


def _mlp_body(ga1_ref, gb1_ref, ga2_ref, gb2_ref, hb1_ref, p1_ref,
              hb2_ref, p2_ref, w1a_ref, w1b_ref, b1_ref,
              w2_ref, b2_ref, w3_ref, b3_ref, out_ref):
    e1 = _sel(ga1_ref, gb1_ref, hb1_ref[...], p1_ref[...])
    e2 = _sel(ga2_ref, gb2_ref, hb2_ref[...], p2_ref[...])
    h = jnp.dot(e1, w1a_ref[...], preferred_element_type=jnp.float32)
    h += jnp.dot(e2, w1b_ref[...], preferred_element_type=jnp.float32)
    h = jnp.maximum(h + b1_ref[...], 0.0)
    h = jnp.maximum(
        jnp.dot(h, w2_ref[...], preferred_element_type=jnp.float32)
        + b2_ref[...], 0.0)
    out_ref[...] = jnp.maximum(
        jnp.dot(h, w3_ref[...], preferred_element_type=jnp.float32)
        + b3_ref[...], 0.0)


def _full(shape):
    return pl.BlockSpec(shape, lambda i: (0,) * len(shape))


@functools.lru_cache(maxsize=None)
def _mlp():
    return pl.pallas_call(
        _mlp_body,
        grid=(B // BLK,),
        in_specs=[
            pl.BlockSpec((BLK, 2 * D), lambda i: (i, 0)),
            pl.BlockSpec((BLK, 2 * D), lambda i: (i, 0)),
            pl.BlockSpec((BLK, 2 * D), lambda i: (i, 0)),
            pl.BlockSpec((BLK, 2 * D), lambda i: (i, 0)),
            pl.BlockSpec((BLK, 1), lambda i: (i, 0)),
            pl.BlockSpec((BLK, 1), lambda i: (i, 0)),
            pl.BlockSpec((BLK, 1), lambda i: (i, 0)),
            pl.BlockSpec((BLK, 1), lambda i: (i, 0)),
            _full((D, 256)),
            _full((D, 256)),
            _full((1, 256)),
            _full((256, 128)),
            _full((1, 128)),
            _full((128, 64)),
            _full((1, 64)),
        ],
        out_specs=pl.BlockSpec((BLK, 64), lambda i: (i, 0)),
        out_shape=jax.ShapeDtypeStruct((B, 64), jnp.float32),
    )


def _split_idx(idx):
    a = jnp.minimum(idx, V // 2 - 1) // 2
    b = (jnp.maximum(idx, V // 2) - V // 2) // 2
    return (a.reshape(NW, NJ, ICH), b.reshape(NW, NJ, ICH),
            (idx >= V // 2).astype(jnp.float32)[:, None],
            (idx % 2).astype(jnp.float32)[:, None])


def kernel(user_id, item_id, emb_user, emb_item, W1, b1, W2, b2, W3, b3):
    uid = user_id.astype(jnp.int32)
    iid = item_id.astype(jnp.int32)
    ua1, ub1, hb1, p1 = _split_idx(uid)
    ia2, ib2, hb2, p2 = _split_idx(iid)
    ga1, gb1, ga2, gb2 = _sc_gather()(
        ua1, ub1, ia2, ib2,
        emb_user[:V // 2].reshape(V // 4, 2 * D),
        emb_user[V // 2:].reshape(V // 4, 2 * D),
        emb_item[:V // 2].reshape(V // 4, 2 * D),
        emb_item[V // 2:].reshape(V // 4, 2 * D))
    return _mlp()(ga1, gb1, ga2, gb2, hb1, p1, hb2, p2,
                  W1[:, :D].T, W1[:, D:].T, b1[None, :],
                  W2.T, b2[None, :], W3.T, b3[None, :])


# speedup vs baseline: 2.5648x; 2.5648x over previous
import functools

import jax
import jax.numpy as jnp
from jax import lax
from jax.experimental import pallas as pl
from jax.experimental.pallas import tpu as pltpu
from jax.experimental.pallas import tpu_sc as plsc

B = 16384
V = 1000000
D = 64

NC, NS = 2, 16
NW = NC * NS
BPW = B // NW
ICH = 128
NJ = BPW // ICH


def _sc_gather_body(uid_hbm, iid_hbm, ut_hbm, it_hbm, g1_hbm, g2_hbm,
                    uidx, iidx, rows, sem):
    wid = lax.axis_index("s") * NC + lax.axis_index("c")
    base = wid * BPW
    pltpu.sync_copy(uid_hbm.at[wid], uidx)
    pltpu.sync_copy(iid_hbm.at[wid], iidx)
    for idxv, tbl, dst in ((uidx, ut_hbm, g1_hbm), (iidx, it_hbm, g2_hbm)):
        copies = [
            pltpu.async_copy(tbl.at[idxv.at[j]],
                             rows.at[pl.ds(j * ICH, ICH)], sem)
            for j in range(NJ)
        ]
        for c in copies:
            c.wait()
        pltpu.sync_copy(rows, dst.at[pl.ds(base, BPW)])


@functools.lru_cache(maxsize=None)
def _sc_gather():
    return pl.kernel(
        _sc_gather_body,
        out_type=(
            jax.ShapeDtypeStruct((B, 2 * D), jnp.float32),
            jax.ShapeDtypeStruct((B, 2 * D), jnp.float32),
        ),
        mesh=plsc.VectorSubcoreMesh(core_axis_name="c", subcore_axis_name="s"),
        scratch_types=[
            pltpu.VMEM((NJ, ICH), jnp.int32),
            pltpu.VMEM((NJ, ICH), jnp.int32),
            pltpu.VMEM((BPW, 2 * D), jnp.float32),
            pltpu.SemaphoreType.DMA,
        ],
    )


BLK = 2048


def _mlp_body(g1_ref, g2_ref, p1_ref, p2_ref, w1a_ref, w1b_ref, b1_ref,
              w2_ref, b2_ref, w3_ref, b3_ref, out_ref):
    p1 = p1_ref[...]
    p2 = p2_ref[...]
    e1 = g1_ref[:, :D] * (1.0 - p1) + g1_ref[:, D:] * p1
    e2 = g2_ref[:, :D] * (1.0 - p2) + g2_ref[:, D:] * p2
    h = jnp.dot(e1, w1a_ref[...], preferred_element_type=jnp.float32)
    h += jnp.dot(e2, w1b_ref[...], preferred_element_type=jnp.float32)
    h = jnp.maximum(h + b1_ref[...], 0.0)
    h = jnp.maximum(
        jnp.dot(h, w2_ref[...], preferred_element_type=jnp.float32)
        + b2_ref[...], 0.0)
    out_ref[...] = jnp.maximum(
        jnp.dot(h, w3_ref[...], preferred_element_type=jnp.float32)
        + b3_ref[...], 0.0)


def _full(shape):
    return pl.BlockSpec(shape, lambda i: (0,) * len(shape))


@functools.lru_cache(maxsize=None)
def _mlp():
    return pl.pallas_call(
        _mlp_body,
        grid=(B // BLK,),
        in_specs=[
            pl.BlockSpec((BLK, 2 * D), lambda i: (i, 0)),
            pl.BlockSpec((BLK, 2 * D), lambda i: (i, 0)),
            pl.BlockSpec((BLK, 1), lambda i: (i, 0)),
            pl.BlockSpec((BLK, 1), lambda i: (i, 0)),
            _full((D, 256)),
            _full((D, 256)),
            _full((1, 256)),
            _full((256, 128)),
            _full((1, 128)),
            _full((128, 64)),
            _full((1, 64)),
        ],
        out_specs=pl.BlockSpec((BLK, 64), lambda i: (i, 0)),
        out_shape=jax.ShapeDtypeStruct((B, 64), jnp.float32),
    )


def kernel(user_id, item_id, emb_user, emb_item, W1, b1, W2, b2, W3, b3):
    uid = user_id.astype(jnp.int32)
    iid = item_id.astype(jnp.int32)
    upair = (uid // 2).reshape(NW, NJ, ICH)
    ipair = (iid // 2).reshape(NW, NJ, ICH)
    g1, g2 = _sc_gather()(upair, ipair,
                          emb_user.reshape(V // 2, 2 * D),
                          emb_item.reshape(V // 2, 2 * D))
    p1 = (uid % 2).astype(jnp.float32)[:, None]
    p2 = (iid % 2).astype(jnp.float32)[:, None]
    return _mlp()(g1, g2, p1, p2, W1[:, :D].T, W1[:, D:].T, b1[None, :],
                  W2.T, b2[None, :], W3.T, b3[None, :])
